# baseline (speedup 1.0000x reference)
import functools

import jax
import jax.numpy as jnp
from jax import lax
from jax.experimental import pallas as pl
from jax.experimental.pallas import tpu as pltpu
from jax.experimental.pallas import tpu_sc as plsc

N_GENES_K = 20000
N_TFS_K = 2048
N_CONN_K = 65536
BATCH_K = 1024

_NC = 2
_NS = 16
_NW = _NC * _NS
_EDGE_SPLIT = 2
_ROW_GROUPS = _NW // _EDGE_SPLIT
_ROWS_PER_W = BATCH_K // _ROW_GROUPS
_EDGES_PER_W = N_CONN_K // _EDGE_SPLIT
_LANES = 16


def _sc_spmm_body(x_hbm, pk_hbm, ew_hbm, out_hbm, pk_v, ew_v, xrow_v, acc_v):
    c = lax.axis_index("c")
    s = lax.axis_index("s")
    wid = s * _NC + c
    half = wid // _ROW_GROUPS
    rgrp = wid % _ROW_GROUPS

    e0 = half * _EDGES_PER_W
    pltpu.sync_copy(pk_hbm.at[pl.ds(e0, _EDGES_PER_W)], pk_v)
    pltpu.sync_copy(ew_hbm.at[pl.ds(e0, _EDGES_PER_W)], ew_v)

    zeros16 = jnp.zeros((_LANES,), jnp.float32)

    def row_body(i, carry):
        b = rgrp * _ROWS_PER_W + i
        pltpu.sync_copy(x_hbm.at[b], xrow_v)

        def zero_body(j, carry2):
            acc_v[pl.ds(j * _LANES, _LANES)] = zeros16
            return carry2

        lax.fori_loop(0, N_TFS_K // _LANES, zero_body, 0, unroll=8)

        def edge_body(e, carry2):
            pk = pk_v[pl.ds(e * _LANES, _LANES)]
            g = lax.shift_right_logical(pk, 11)
            t = lax.bitwise_and(pk, 2047)
            w = ew_v[pl.ds(e * _LANES, _LANES)]
            vals = plsc.load_gather(xrow_v, [g])
            plsc.addupdate_scatter(acc_v, [t], vals * w)
            return carry2

        lax.fori_loop(0, _EDGES_PER_W // _LANES, edge_body, 0, unroll=4)
        pltpu.sync_copy(acc_v, out_hbm.at[half, b])
        return carry

    lax.fori_loop(0, _ROWS_PER_W, row_body, 0)


@jax.jit
def _sc_spmm(x, packed, ew):
    mesh = plsc.VectorSubcoreMesh(core_axis_name="c", subcore_axis_name="s",
                                  num_cores=_NC, num_subcores=_NS)
    return pl.kernel(
        _sc_spmm_body,
        out_type=jax.ShapeDtypeStruct((_EDGE_SPLIT, BATCH_K, N_TFS_K),
                                      jnp.float32),
        mesh=mesh,
        scratch_types=[
            pltpu.VMEM((_EDGES_PER_W,), jnp.int32),
            pltpu.VMEM((_EDGES_PER_W,), jnp.float32),
            pltpu.VMEM((N_GENES_K,), jnp.float32),
            pltpu.VMEM((N_TFS_K,), jnp.float32),
        ],
        compiler_params=pltpu.CompilerParams(needs_layout_passes=False),
    )(x, packed, ew)


def _prelu(h, a):
    return jnp.maximum(h, 0.0) + a * jnp.minimum(h, 0.0)


_BB = 128
_GB = 2048
_NB = BATCH_K // _BB
_NG = (N_GENES_K + _GB - 1) // _GB


def _mlp_body(p_ref, pe_ref, w1_ref, b1_ref, w2_ref, b2_ref, w3_ref, b3_ref,
              w4_ref, b4_ref, a_ref, out_ref, h_scr):
    j = pl.program_id(1)

    @pl.when(j == 0)
    def _encode():
        tf = p_ref[0] + p_ref[1]
        a0 = a_ref[0, 0]
        a1 = a_ref[0, 1]
        a2 = a_ref[0, 2]
        a3 = a_ref[0, 3]
        h = _prelu(tf, a0)
        h = lax.dot_general(h, w1_ref[...], (((1,), (1,)), ((), ())),
                            preferred_element_type=jnp.float32,
                            precision=lax.Precision.HIGHEST) + b1_ref[...]
        h = _prelu(h, a1)
        h = lax.dot_general(h, w2_ref[...], (((1,), (1,)), ((), ())),
                            preferred_element_type=jnp.float32,
                            precision=lax.Precision.HIGHEST) + b2_ref[...]
        h = _prelu(h, a2)
        h = h + pe_ref[...]
        h = lax.dot_general(h, w3_ref[...], (((1,), (1,)), ((), ())),
                            preferred_element_type=jnp.float32,
                            precision=lax.Precision.HIGHEST) + b3_ref[...]
        h_scr[...] = _prelu(h, a3)

    out_ref[...] = lax.dot_general(
        h_scr[...], w4_ref[...], (((1,), (1,)), ((), ())),
        preferred_element_type=jnp.float32,
        precision=lax.Precision.HIGHEST) + b4_ref[...]


@jax.jit
def _tc_mlp(partials, pe, W1, b1, W2, b2, W3, b3, W4, b4, a_all):
    grid = (_NB, _NG)
    return pl.pallas_call(
        _mlp_body,
        grid=grid,
        in_specs=[
            pl.BlockSpec((_EDGE_SPLIT, _BB, N_TFS_K), lambda i, j: (0, i, 0)),
            pl.BlockSpec((_BB, 64), lambda i, j: (i, 0)),
            pl.BlockSpec((64, N_TFS_K), lambda i, j: (0, 0)),
            pl.BlockSpec((1, 64), lambda i, j: (0, 0)),
            pl.BlockSpec((64, 64), lambda i, j: (0, 0)),
            pl.BlockSpec((1, 64), lambda i, j: (0, 0)),
            pl.BlockSpec((64, 64), lambda i, j: (0, 0)),
            pl.BlockSpec((1, 64), lambda i, j: (0, 0)),
            pl.BlockSpec((_GB, 64), lambda i, j: (j, 0)),
            pl.BlockSpec((1, _GB), lambda i, j: (0, j)),
            pl.BlockSpec((1, 4), lambda i, j: (0, 0)),
        ],
        out_specs=pl.BlockSpec((_BB, _GB), lambda i, j: (i, j)),
        out_shape=jax.ShapeDtypeStruct((BATCH_K, N_GENES_K), jnp.float32),
        scratch_shapes=[pltpu.VMEM((_BB, 64), jnp.float32)],
    )(partials, pe, W1, b1, W2, b2, W3, b3, W4, b4, a_all)


def kernel(x, pert, gene_indices, tf_indices, edge_weights, pert_table,
           W1, b1, W2, b2, W3, b3, W4, b4, a0, a1, a2, a3):
    packed = (gene_indices.astype(jnp.int32) << 11) | tf_indices.astype(jnp.int32)
    partials = _sc_spmm(x, packed, edge_weights)
    pe = jnp.take(pert_table, pert, axis=0)
    a_all = jnp.stack([a0[0], a1[0], a2[0], a3[0]]).reshape(1, 4)
    return _tc_mlp(partials, pe, W1, b1.reshape(1, 64), W2, b2.reshape(1, 64),
                   W3, b3.reshape(1, 64), W4, b4.reshape(1, 20000), a_all)


# speedup vs baseline: 1.0758x; 1.0758x over previous
import functools

import jax
import jax.numpy as jnp
from jax import lax
from jax.experimental import pallas as pl
from jax.experimental.pallas import tpu as pltpu
from jax.experimental.pallas import tpu_sc as plsc

N_GENES_K = 20000
N_TFS_K = 2048
N_CONN_K = 65536
BATCH_K = 1024

_NC = 2
_NS = 16
_NW = _NC * _NS
_EDGE_SPLIT = 2
_ROW_GROUPS = _NW // _EDGE_SPLIT
_ROWS_PER_W = BATCH_K // _ROW_GROUPS
_EDGES_PER_W = N_CONN_K // _EDGE_SPLIT
_LANES = 16


def _sc_spmm_body(x_hbm, pk_hbm, ew_hbm, out_hbm, pk_v, ew_v,
                  xrow_a, xrow_b, acc_a, acc_b,
                  sem_xa, sem_xb, sem_oa, sem_ob):
    c = lax.axis_index("c")
    s = lax.axis_index("s")
    wid = s * _NC + c
    half = wid // _ROW_GROUPS
    rgrp = wid % _ROW_GROUPS
    base = rgrp * _ROWS_PER_W

    e0 = half * _EDGES_PER_W
    pltpu.sync_copy(pk_hbm.at[pl.ds(e0, _EDGES_PER_W)], pk_v)
    pltpu.sync_copy(ew_hbm.at[pl.ds(e0, _EDGES_PER_W)], ew_v)

    zeros16 = jnp.zeros((_LANES,), jnp.float32)

    def zero_acc(acc):
        def zero_body(j, carry2):
            acc[pl.ds(j * _LANES, _LANES)] = zeros16
            return carry2
        lax.fori_loop(0, N_TFS_K // _LANES, zero_body, 0, unroll=16)

    def accumulate(xrow, acc):
        def edge_body(e, carry2):
            pk = pk_v[pl.ds(e * _LANES, _LANES)]
            g = lax.shift_right_logical(pk, 11)
            t = lax.bitwise_and(pk, 2047)
            w = ew_v[pl.ds(e * _LANES, _LANES)]
            vals = plsc.load_gather(xrow, [g])
            plsc.addupdate_scatter(acc, [t], vals * w)
            return carry2
        lax.fori_loop(0, _EDGES_PER_W // _LANES, edge_body, 0, unroll=8)

    pltpu.async_copy(x_hbm.at[base], xrow_a, sem_xa)

    def pair_body(i, carry):
        b0 = base + 2 * i
        b1 = b0 + 1
        pltpu.async_copy(x_hbm.at[b1], xrow_b, sem_xb)

        @pl.when(i > 0)
        def _wait_oa():
            pltpu.make_async_copy(acc_a, out_hbm.at[half, b0 - 2], sem_oa).wait()

        zero_acc(acc_a)
        pltpu.make_async_copy(x_hbm.at[b0], xrow_a, sem_xa).wait()
        accumulate(xrow_a, acc_a)
        pltpu.async_copy(acc_a, out_hbm.at[half, b0], sem_oa)

        @pl.when(i < _ROWS_PER_W // 2 - 1)
        def _next_even():
            pltpu.async_copy(x_hbm.at[b0 + 2], xrow_a, sem_xa)

        @pl.when(i > 0)
        def _wait_ob():
            pltpu.make_async_copy(acc_b, out_hbm.at[half, b1 - 2], sem_ob).wait()

        zero_acc(acc_b)
        pltpu.make_async_copy(x_hbm.at[b1], xrow_b, sem_xb).wait()
        accumulate(xrow_b, acc_b)
        pltpu.async_copy(acc_b, out_hbm.at[half, b1], sem_ob)
        return carry

    lax.fori_loop(0, _ROWS_PER_W // 2, pair_body, 0)
    last = base + _ROWS_PER_W
    pltpu.make_async_copy(acc_a, out_hbm.at[half, last - 2], sem_oa).wait()
    pltpu.make_async_copy(acc_b, out_hbm.at[half, last - 1], sem_ob).wait()


@jax.jit
def _sc_spmm(x, packed, ew):
    mesh = plsc.VectorSubcoreMesh(core_axis_name="c", subcore_axis_name="s",
                                  num_cores=_NC, num_subcores=_NS)
    return pl.kernel(
        _sc_spmm_body,
        out_type=jax.ShapeDtypeStruct((_EDGE_SPLIT, BATCH_K, N_TFS_K),
                                      jnp.float32),
        mesh=mesh,
        scratch_types=[
            pltpu.VMEM((_EDGES_PER_W,), jnp.int32),
            pltpu.VMEM((_EDGES_PER_W,), jnp.float32),
            pltpu.VMEM((N_GENES_K,), jnp.float32),
            pltpu.VMEM((N_GENES_K,), jnp.float32),
            pltpu.VMEM((N_TFS_K,), jnp.float32),
            pltpu.VMEM((N_TFS_K,), jnp.float32),
            pltpu.SemaphoreType.DMA,
            pltpu.SemaphoreType.DMA,
            pltpu.SemaphoreType.DMA,
            pltpu.SemaphoreType.DMA,
        ],
        compiler_params=pltpu.CompilerParams(needs_layout_passes=False),
    )(x, packed, ew)


def _prelu(h, a):
    return jnp.maximum(h, 0.0) + a * jnp.minimum(h, 0.0)


_BB = 128
_GB = 2048
_NB = BATCH_K // _BB
_NG = (N_GENES_K + _GB - 1) // _GB


def _mlp_body(p_ref, pe_ref, w1_ref, b1_ref, w2_ref, b2_ref, w3_ref, b3_ref,
              w4_ref, b4_ref, a_ref, out_ref, h_scr):
    j = pl.program_id(1)

    @pl.when(j == 0)
    def _encode():
        tf = p_ref[0] + p_ref[1]
        a0 = a_ref[0, 0]
        a1 = a_ref[0, 1]
        a2 = a_ref[0, 2]
        a3 = a_ref[0, 3]
        h = _prelu(tf, a0)
        h = lax.dot_general(h.astype(jnp.bfloat16), w1_ref[...],
                            (((1,), (1,)), ((), ())),
                            preferred_element_type=jnp.float32) + b1_ref[...]
        h = _prelu(h, a1)
        h = lax.dot_general(h.astype(jnp.bfloat16), w2_ref[...],
                            (((1,), (1,)), ((), ())),
                            preferred_element_type=jnp.float32) + b2_ref[...]
        h = _prelu(h, a2)
        h = h + pe_ref[...]
        h = lax.dot_general(h.astype(jnp.bfloat16), w3_ref[...],
                            (((1,), (1,)), ((), ())),
                            preferred_element_type=jnp.float32) + b3_ref[...]
        h_scr[...] = _prelu(h, a3).astype(jnp.bfloat16)

    out_ref[...] = lax.dot_general(
        h_scr[...], w4_ref[...], (((1,), (1,)), ((), ())),
        preferred_element_type=jnp.float32) + b4_ref[...]


@jax.jit
def _tc_mlp(partials, pe, W1, b1, W2, b2, W3, b3, W4, b4, a_all):
    grid = (_NB, _NG)
    return pl.pallas_call(
        _mlp_body,
        grid=grid,
        in_specs=[
            pl.BlockSpec((_EDGE_SPLIT, _BB, N_TFS_K), lambda i, j: (0, i, 0)),
            pl.BlockSpec((_BB, 64), lambda i, j: (i, 0)),
            pl.BlockSpec((64, N_TFS_K), lambda i, j: (0, 0)),
            pl.BlockSpec((1, 64), lambda i, j: (0, 0)),
            pl.BlockSpec((64, 64), lambda i, j: (0, 0)),
            pl.BlockSpec((1, 64), lambda i, j: (0, 0)),
            pl.BlockSpec((64, 64), lambda i, j: (0, 0)),
            pl.BlockSpec((1, 64), lambda i, j: (0, 0)),
            pl.BlockSpec((_GB, 64), lambda i, j: (j, 0)),
            pl.BlockSpec((1, _GB), lambda i, j: (0, j)),
            pl.BlockSpec((1, 4), lambda i, j: (0, 0)),
        ],
        out_specs=pl.BlockSpec((_BB, _GB), lambda i, j: (i, j)),
        out_shape=jax.ShapeDtypeStruct((BATCH_K, N_GENES_K), jnp.float32),
        scratch_shapes=[pltpu.VMEM((_BB, 64), jnp.bfloat16)],
    )(partials, pe, W1, b1, W2, b2, W3, b3, W4, b4, a_all)


def kernel(x, pert, gene_indices, tf_indices, edge_weights, pert_table,
           W1, b1, W2, b2, W3, b3, W4, b4, a0, a1, a2, a3):
    packed = (gene_indices.astype(jnp.int32) << 11) | tf_indices.astype(jnp.int32)
    partials = _sc_spmm(x, packed, edge_weights)
    pe = jnp.take(pert_table, pert, axis=0)
    a_all = jnp.stack([a0[0], a1[0], a2[0], a3[0]]).reshape(1, 4)
    return _tc_mlp(partials, pe, W1.astype(jnp.bfloat16), b1.reshape(1, 64),
                   W2.astype(jnp.bfloat16), b2.reshape(1, 64),
                   W3.astype(jnp.bfloat16), b3.reshape(1, 64),
                   W4.astype(jnp.bfloat16), b4.reshape(1, 20000), a_all)


# speedup vs baseline: 2.8296x; 2.6302x over previous
import functools

import jax
import jax.numpy as jnp
from jax import lax
from jax.experimental import pallas as pl
from jax.experimental.pallas import tpu as pltpu
from jax.experimental.pallas import tpu_sc as plsc

N_GENES_K = 20000
N_TFS_K = 2048
N_CONN_K = 65536
BATCH_K = 1024

_NC = 2
_NS = 16
_NW = _NC * _NS
_EDGE_SPLIT = 2
_ROW_GROUPS = _NW // _EDGE_SPLIT
_ROWS_PER_W = BATCH_K // _ROW_GROUPS
_EDGES_PER_W = N_CONN_K // _EDGE_SPLIT
_LANES = 16


def _sc_spmm_body(x_hbm, pk_hbm, ew_hbm, out_hbm, pk_v, ew_v,
                  xrow_a, xrow_b, acc_a, acc_b,
                  sem_xa, sem_xb, sem_oa, sem_ob):
    c = lax.axis_index("c")
    s = lax.axis_index("s")
    wid = s * _NC + c
    half = wid // _ROW_GROUPS
    rgrp = wid % _ROW_GROUPS
    base = rgrp * _ROWS_PER_W

    e0 = half * _EDGES_PER_W
    pltpu.sync_copy(pk_hbm.at[pl.ds(e0, _EDGES_PER_W)], pk_v)
    pltpu.sync_copy(ew_hbm.at[pl.ds(e0, _EDGES_PER_W)], ew_v)

    zeros16 = jnp.zeros((_LANES,), jnp.float32)

    def zero_acc(acc):
        @plsc.parallel_loop(0, N_TFS_K, step=_LANES, unroll=8)
        def zero_body(j):
            acc[pl.ds(j, _LANES)] = zeros16

    def accumulate(xrow, acc):
        @plsc.parallel_loop(0, _EDGES_PER_W, step=_LANES, unroll=8)
        def edge_body(e):
            pk = pk_v[pl.ds(e, _LANES)]
            g = lax.shift_right_logical(pk, 11)
            t = lax.bitwise_and(pk, 2047)
            w = ew_v[pl.ds(e, _LANES)]
            vals = plsc.load_gather(xrow, [g])
            plsc.addupdate_scatter(acc, [t], vals * w)

    pltpu.async_copy(x_hbm.at[base], xrow_a, sem_xa)

    def pair_body(i, carry):
        b0 = base + 2 * i
        b1 = b0 + 1
        pltpu.async_copy(x_hbm.at[b1], xrow_b, sem_xb)

        @pl.when(i > 0)
        def _wait_oa():
            pltpu.make_async_copy(acc_a, out_hbm.at[half, b0 - 2], sem_oa).wait()

        zero_acc(acc_a)
        pltpu.make_async_copy(x_hbm.at[b0], xrow_a, sem_xa).wait()
        accumulate(xrow_a, acc_a)
        pltpu.async_copy(acc_a, out_hbm.at[half, b0], sem_oa)

        @pl.when(i < _ROWS_PER_W // 2 - 1)
        def _next_even():
            pltpu.async_copy(x_hbm.at[b0 + 2], xrow_a, sem_xa)

        @pl.when(i > 0)
        def _wait_ob():
            pltpu.make_async_copy(acc_b, out_hbm.at[half, b1 - 2], sem_ob).wait()

        zero_acc(acc_b)
        pltpu.make_async_copy(x_hbm.at[b1], xrow_b, sem_xb).wait()
        accumulate(xrow_b, acc_b)
        pltpu.async_copy(acc_b, out_hbm.at[half, b1], sem_ob)
        return carry

    lax.fori_loop(0, _ROWS_PER_W // 2, pair_body, 0)
    last = base + _ROWS_PER_W
    pltpu.make_async_copy(acc_a, out_hbm.at[half, last - 2], sem_oa).wait()
    pltpu.make_async_copy(acc_b, out_hbm.at[half, last - 1], sem_ob).wait()


@jax.jit
def _sc_spmm(x, packed, ew):
    mesh = plsc.VectorSubcoreMesh(core_axis_name="c", subcore_axis_name="s",
                                  num_cores=_NC, num_subcores=_NS)
    return pl.kernel(
        _sc_spmm_body,
        out_type=jax.ShapeDtypeStruct((_EDGE_SPLIT, BATCH_K, N_TFS_K),
                                      jnp.float32),
        mesh=mesh,
        scratch_types=[
            pltpu.VMEM((_EDGES_PER_W,), jnp.int32),
            pltpu.VMEM((_EDGES_PER_W,), jnp.float32),
            pltpu.VMEM((N_GENES_K,), jnp.float32),
            pltpu.VMEM((N_GENES_K,), jnp.float32),
            pltpu.VMEM((N_TFS_K,), jnp.float32),
            pltpu.VMEM((N_TFS_K,), jnp.float32),
            pltpu.SemaphoreType.DMA,
            pltpu.SemaphoreType.DMA,
            pltpu.SemaphoreType.DMA,
            pltpu.SemaphoreType.DMA,
        ],
        compiler_params=pltpu.CompilerParams(needs_layout_passes=False),
    )(x, packed, ew)


def _prelu(h, a):
    return jnp.maximum(h, 0.0) + a * jnp.minimum(h, 0.0)


_BB = 128
_GB = 2048
_NB = BATCH_K // _BB
_NG = (N_GENES_K + _GB - 1) // _GB


def _mlp_body(p_ref, pe_ref, w1_ref, b1_ref, w2_ref, b2_ref, w3_ref, b3_ref,
              w4_ref, b4_ref, a_ref, out_ref, h_scr):
    j = pl.program_id(1)

    @pl.when(j == 0)
    def _encode():
        tf = p_ref[0] + p_ref[1]
        a0 = a_ref[0, 0]
        a1 = a_ref[0, 1]
        a2 = a_ref[0, 2]
        a3 = a_ref[0, 3]
        h = _prelu(tf, a0)
        h = lax.dot_general(h.astype(jnp.bfloat16), w1_ref[...],
                            (((1,), (1,)), ((), ())),
                            preferred_element_type=jnp.float32) + b1_ref[...]
        h = _prelu(h, a1)
        h = lax.dot_general(h.astype(jnp.bfloat16), w2_ref[...],
                            (((1,), (1,)), ((), ())),
                            preferred_element_type=jnp.float32) + b2_ref[...]
        h = _prelu(h, a2)
        h = h + pe_ref[...]
        h = lax.dot_general(h.astype(jnp.bfloat16), w3_ref[...],
                            (((1,), (1,)), ((), ())),
                            preferred_element_type=jnp.float32) + b3_ref[...]
        h_scr[...] = _prelu(h, a3).astype(jnp.bfloat16)

    out_ref[...] = lax.dot_general(
        h_scr[...], w4_ref[...], (((1,), (1,)), ((), ())),
        preferred_element_type=jnp.float32) + b4_ref[...]


@jax.jit
def _tc_mlp(partials, pe, W1, b1, W2, b2, W3, b3, W4, b4, a_all):
    grid = (_NB, _NG)
    return pl.pallas_call(
        _mlp_body,
        grid=grid,
        in_specs=[
            pl.BlockSpec((_EDGE_SPLIT, _BB, N_TFS_K), lambda i, j: (0, i, 0)),
            pl.BlockSpec((_BB, 64), lambda i, j: (i, 0)),
            pl.BlockSpec((64, N_TFS_K), lambda i, j: (0, 0)),
            pl.BlockSpec((1, 64), lambda i, j: (0, 0)),
            pl.BlockSpec((64, 64), lambda i, j: (0, 0)),
            pl.BlockSpec((1, 64), lambda i, j: (0, 0)),
            pl.BlockSpec((64, 64), lambda i, j: (0, 0)),
            pl.BlockSpec((1, 64), lambda i, j: (0, 0)),
            pl.BlockSpec((_GB, 64), lambda i, j: (j, 0)),
            pl.BlockSpec((1, _GB), lambda i, j: (0, j)),
            pl.BlockSpec((1, 4), lambda i, j: (0, 0)),
        ],
        out_specs=pl.BlockSpec((_BB, _GB), lambda i, j: (i, j)),
        out_shape=jax.ShapeDtypeStruct((BATCH_K, N_GENES_K), jnp.float32),
        scratch_shapes=[pltpu.VMEM((_BB, 64), jnp.bfloat16)],
    )(partials, pe, W1, b1, W2, b2, W3, b3, W4, b4, a_all)


def kernel(x, pert, gene_indices, tf_indices, edge_weights, pert_table,
           W1, b1, W2, b2, W3, b3, W4, b4, a0, a1, a2, a3):
    packed = (gene_indices.astype(jnp.int32) << 11) | tf_indices.astype(jnp.int32)
    partials = _sc_spmm(x, packed, edge_weights)
    pe = jnp.take(pert_table, pert, axis=0)
    a_all = jnp.stack([a0[0], a1[0], a2[0], a3[0]]).reshape(1, 4)
    return _tc_mlp(partials, pe, W1.astype(jnp.bfloat16), b1.reshape(1, 64),
                   W2.astype(jnp.bfloat16), b2.reshape(1, 64),
                   W3.astype(jnp.bfloat16), b3.reshape(1, 64),
                   W4.astype(jnp.bfloat16), b4.reshape(1, 20000), a_all)


# speedup vs baseline: 3.1431x; 1.1108x over previous
import functools

import jax
import jax.numpy as jnp
from jax import lax
from jax.experimental import pallas as pl
from jax.experimental.pallas import tpu as pltpu
from jax.experimental.pallas import tpu_sc as plsc

N_GENES_K = 20000
N_TFS_K = 2048
N_CONN_K = 65536
BATCH_K = 1024

_NC = 2
_NS = 16
_NW = _NC * _NS
_EDGE_SPLIT = 4
_ROW_GROUPS = _NW // _EDGE_SPLIT
_ROWS_PER_W = BATCH_K // _ROW_GROUPS
_EDGES_PER_W = N_CONN_K // _EDGE_SPLIT
_LANES = 16


def _sc_spmm_body(x_hbm, pk_hbm, ew_hbm, out_hbm, pk_v, ew_v,
                  xa0, xa1, xb0, xb1, aa0, aa1, ab0, ab1,
                  sem_xa, sem_xb, sem_oa, sem_ob):
    c = lax.axis_index("c")
    s = lax.axis_index("s")
    wid = s * _NC + c
    q = wid // _ROW_GROUPS
    rgrp = wid % _ROW_GROUPS
    base = rgrp * _ROWS_PER_W

    e0 = q * _EDGES_PER_W
    pltpu.sync_copy(pk_hbm.at[pl.ds(e0, _EDGES_PER_W)], pk_v)
    pltpu.sync_copy(ew_hbm.at[pl.ds(e0, _EDGES_PER_W)], ew_v)

    zeros16 = jnp.zeros((_LANES,), jnp.float32)

    def zero_acc(acc):
        @plsc.parallel_loop(0, N_TFS_K, step=_LANES, unroll=8)
        def zero_body(j):
            acc[pl.ds(j, _LANES)] = zeros16

    def accumulate_pair(x0, x1, a0, a1):
        @plsc.parallel_loop(0, _EDGES_PER_W, step=_LANES, unroll=8)
        def edge_body(e):
            pk = pk_v[pl.ds(e, _LANES)]
            g = lax.shift_right_logical(pk, 11)
            t = lax.bitwise_and(pk, 2047)
            w = ew_v[pl.ds(e, _LANES)]
            v0 = plsc.load_gather(x0, [g])
            plsc.addupdate_scatter(a0, [t], v0 * w)
            v1 = plsc.load_gather(x1, [g])
            plsc.addupdate_scatter(a1, [t], v1 * w)

    pltpu.async_copy(x_hbm.at[base], xa0, sem_xa)
    pltpu.async_copy(x_hbm.at[base + 1], xa1, sem_xa)

    def quad_body(i, carry):
        r0 = base + 4 * i
        pltpu.async_copy(x_hbm.at[r0 + 2], xb0, sem_xb)
        pltpu.async_copy(x_hbm.at[r0 + 3], xb1, sem_xb)

        @pl.when(i > 0)
        def _wait_oa():
            pltpu.make_async_copy(aa0, out_hbm.at[q, r0 - 4], sem_oa).wait()
            pltpu.make_async_copy(aa1, out_hbm.at[q, r0 - 3], sem_oa).wait()

        zero_acc(aa0)
        zero_acc(aa1)
        pltpu.make_async_copy(x_hbm.at[r0], xa0, sem_xa).wait()
        pltpu.make_async_copy(x_hbm.at[r0 + 1], xa1, sem_xa).wait()
        accumulate_pair(xa0, xa1, aa0, aa1)
        pltpu.async_copy(aa0, out_hbm.at[q, r0], sem_oa)
        pltpu.async_copy(aa1, out_hbm.at[q, r0 + 1], sem_oa)

        @pl.when(i < _ROWS_PER_W // 4 - 1)
        def _next_a():
            pltpu.async_copy(x_hbm.at[r0 + 4], xa0, sem_xa)
            pltpu.async_copy(x_hbm.at[r0 + 5], xa1, sem_xa)

        @pl.when(i > 0)
        def _wait_ob():
            pltpu.make_async_copy(ab0, out_hbm.at[q, r0 - 2], sem_ob).wait()
            pltpu.make_async_copy(ab1, out_hbm.at[q, r0 - 1], sem_ob).wait()

        zero_acc(ab0)
        zero_acc(ab1)
        pltpu.make_async_copy(x_hbm.at[r0 + 2], xb0, sem_xb).wait()
        pltpu.make_async_copy(x_hbm.at[r0 + 3], xb1, sem_xb).wait()
        accumulate_pair(xb0, xb1, ab0, ab1)
        pltpu.async_copy(ab0, out_hbm.at[q, r0 + 2], sem_ob)
        pltpu.async_copy(ab1, out_hbm.at[q, r0 + 3], sem_ob)
        return carry

    lax.fori_loop(0, _ROWS_PER_W // 4, quad_body, 0)
    last = base + _ROWS_PER_W
    pltpu.make_async_copy(aa0, out_hbm.at[q, last - 4], sem_oa).wait()
    pltpu.make_async_copy(aa1, out_hbm.at[q, last - 3], sem_oa).wait()
    pltpu.make_async_copy(ab0, out_hbm.at[q, last - 2], sem_ob).wait()
    pltpu.make_async_copy(ab1, out_hbm.at[q, last - 1], sem_ob).wait()


@jax.jit
def _sc_spmm(x, packed, ew):
    mesh = plsc.VectorSubcoreMesh(core_axis_name="c", subcore_axis_name="s",
                                  num_cores=_NC, num_subcores=_NS)
    return pl.kernel(
        _sc_spmm_body,
        out_type=jax.ShapeDtypeStruct((_EDGE_SPLIT, BATCH_K, N_TFS_K),
                                      jnp.float32),
        mesh=mesh,
        scratch_types=[
            pltpu.VMEM((_EDGES_PER_W,), jnp.int32),
            pltpu.VMEM((_EDGES_PER_W,), jnp.float32),
            pltpu.VMEM((N_GENES_K,), jnp.float32),
            pltpu.VMEM((N_GENES_K,), jnp.float32),
            pltpu.VMEM((N_GENES_K,), jnp.float32),
            pltpu.VMEM((N_GENES_K,), jnp.float32),
            pltpu.VMEM((N_TFS_K,), jnp.float32),
            pltpu.VMEM((N_TFS_K,), jnp.float32),
            pltpu.VMEM((N_TFS_K,), jnp.float32),
            pltpu.VMEM((N_TFS_K,), jnp.float32),
            pltpu.SemaphoreType.DMA,
            pltpu.SemaphoreType.DMA,
            pltpu.SemaphoreType.DMA,
            pltpu.SemaphoreType.DMA,
        ],
        compiler_params=pltpu.CompilerParams(needs_layout_passes=False),
    )(x, packed, ew)


def _prelu(h, a):
    return jnp.maximum(h, 0.0) + a * jnp.minimum(h, 0.0)


_BB = 128
_GB = 2048
_NB = BATCH_K // _BB
_NG = (N_GENES_K + _GB - 1) // _GB


def _mlp_body(p_ref, pe_ref, w1_ref, b1_ref, w2_ref, b2_ref, w3_ref, b3_ref,
              w4_ref, b4_ref, a_ref, out_ref, h_scr):
    j = pl.program_id(1)

    @pl.when(j == 0)
    def _encode():
        tf = (p_ref[0] + p_ref[1]) + (p_ref[2] + p_ref[3])
        a0 = a_ref[0, 0]
        a1 = a_ref[0, 1]
        a2 = a_ref[0, 2]
        a3 = a_ref[0, 3]
        h = _prelu(tf, a0)
        h = lax.dot_general(h.astype(jnp.bfloat16), w1_ref[...],
                            (((1,), (1,)), ((), ())),
                            preferred_element_type=jnp.float32) + b1_ref[...]
        h = _prelu(h, a1)
        h = lax.dot_general(h.astype(jnp.bfloat16), w2_ref[...],
                            (((1,), (1,)), ((), ())),
                            preferred_element_type=jnp.float32) + b2_ref[...]
        h = _prelu(h, a2)
        h = h + pe_ref[...]
        h = lax.dot_general(h.astype(jnp.bfloat16), w3_ref[...],
                            (((1,), (1,)), ((), ())),
                            preferred_element_type=jnp.float32) + b3_ref[...]
        h_scr[...] = _prelu(h, a3).astype(jnp.bfloat16)

    out_ref[...] = lax.dot_general(
        h_scr[...], w4_ref[...], (((1,), (1,)), ((), ())),
        preferred_element_type=jnp.float32) + b4_ref[...]


@jax.jit
def _tc_mlp(partials, pe, W1, b1, W2, b2, W3, b3, W4, b4, a_all):
    grid = (_NB, _NG)
    return pl.pallas_call(
        _mlp_body,
        grid=grid,
        in_specs=[
            pl.BlockSpec((_EDGE_SPLIT, _BB, N_TFS_K), lambda i, j: (0, i, 0)),
            pl.BlockSpec((_BB, 64), lambda i, j: (i, 0)),
            pl.BlockSpec((64, N_TFS_K), lambda i, j: (0, 0)),
            pl.BlockSpec((1, 64), lambda i, j: (0, 0)),
            pl.BlockSpec((64, 64), lambda i, j: (0, 0)),
            pl.BlockSpec((1, 64), lambda i, j: (0, 0)),
            pl.BlockSpec((64, 64), lambda i, j: (0, 0)),
            pl.BlockSpec((1, 64), lambda i, j: (0, 0)),
            pl.BlockSpec((_GB, 64), lambda i, j: (j, 0)),
            pl.BlockSpec((1, _GB), lambda i, j: (0, j)),
            pl.BlockSpec((1, 4), lambda i, j: (0, 0)),
        ],
        out_specs=pl.BlockSpec((_BB, _GB), lambda i, j: (i, j)),
        out_shape=jax.ShapeDtypeStruct((BATCH_K, N_GENES_K), jnp.float32),
        scratch_shapes=[pltpu.VMEM((_BB, 64), jnp.bfloat16)],
    )(partials, pe, W1, b1, W2, b2, W3, b3, W4, b4, a_all)


def kernel(x, pert, gene_indices, tf_indices, edge_weights, pert_table,
           W1, b1, W2, b2, W3, b3, W4, b4, a0, a1, a2, a3):
    packed = (gene_indices.astype(jnp.int32) << 11) | tf_indices.astype(jnp.int32)
    partials = _sc_spmm(x, packed, edge_weights)
    pe = jnp.take(pert_table, pert, axis=0)
    a_all = jnp.stack([a0[0], a1[0], a2[0], a3[0]]).reshape(1, 4)
    return _tc_mlp(partials, pe, W1.astype(jnp.bfloat16), b1.reshape(1, 64),
                   W2.astype(jnp.bfloat16), b2.reshape(1, 64),
                   W3.astype(jnp.bfloat16), b3.reshape(1, 64),
                   W4.astype(jnp.bfloat16), b4.reshape(1, 20000), a_all)
